# baseline (speedup 1.0000x reference)
import jax
import jax.numpy as jnp
from jax.experimental import pallas as pl
from jax.experimental.pallas import tpu as pltpu

_LANES = 128
_CR = 4096
_K = 8


def _ring_copy_kernel(idx_ref, a_ref, o_ref, bufs, *sems):
    rows = a_ref.shape[0]
    nch = rows // _CR
    rsem = sems[:_K]
    wsem = sems[_K:]
    idx = idx_ref[0]

    def rd(i):
        s = i % _K
        return pltpu.make_async_copy(
            a_ref.at[pl.ds(i * _CR, _CR), :], bufs.at[s], rsem[s])

    def wr(i):
        s = i % _K
        return pltpu.make_async_copy(
            bufs.at[s], o_ref.at[pl.ds(i * _CR, _CR), :], wsem[s])

    for i in range(min(_K, nch)):
        rd(i).start()

    for i in range(nch):
        s = i % _K
        rd(i).wait()

        base = i * _CR * _LANES
        contains = jnp.logical_and(idx + 1 >= base,
                                   idx < base + _CR * _LANES)

        @pl.when(contains)
        def _fix(s=s, base=base):
            rr = jax.lax.broadcasted_iota(jnp.int32, (_CR, _LANES), 0)
            cc = jax.lax.broadcasted_iota(jnp.int32, (_CR, _LANES), 1)
            flat = base + rr * _LANES + cc
            mask = jnp.logical_or(flat == idx, flat == idx + 1)
            bufs[s] = jnp.where(mask, jnp.float32(0), bufs[s])

        wr(i).start()
        if i + _K < nch:
            wr(i).wait()
            rd(i + _K).start()

    for i in range(max(nch - _K, 0), nch):
        wr(i).wait()


def kernel(a, idx):
    n = a.shape[0]
    rows = n // _LANES
    idx32 = idx.astype(jnp.int32)
    a2 = a.reshape(rows, _LANES)
    out = pl.pallas_call(
        _ring_copy_kernel,
        out_shape=jax.ShapeDtypeStruct((rows, _LANES), a.dtype),
        in_specs=[
            pl.BlockSpec(memory_space=pltpu.SMEM),
            pl.BlockSpec(memory_space=pltpu.MemorySpace.HBM),
        ],
        out_specs=pl.BlockSpec(memory_space=pltpu.MemorySpace.HBM),
        scratch_shapes=[pltpu.VMEM((_K, _CR, _LANES), jnp.float32)]
        + [pltpu.SemaphoreType.DMA] * (2 * _K),
    )(idx32, a2)
    return out.reshape(n)


# speedup vs baseline: 1.0693x; 1.0693x over previous
import jax
import jax.numpy as jnp
from jax.experimental import pallas as pl
from jax.experimental.pallas import tpu as pltpu

_LANES = 128
_CR = 2048
_K = 12
_WLAG = 4


def _ring_copy_kernel(idx_ref, a_ref, o_ref, bufs, *sems):
    rows = a_ref.shape[0]
    nch = rows // _CR
    rsem = sems[:_K]
    wsem = sems[_K:]
    idx = idx_ref[0]

    def rd(i):
        s = i % _K
        return pltpu.make_async_copy(
            a_ref.at[pl.ds(i * _CR, _CR), :], bufs.at[s], rsem[s])

    def wr(i):
        s = i % _K
        return pltpu.make_async_copy(
            bufs.at[s], o_ref.at[pl.ds(i * _CR, _CR), :], wsem[s])

    for i in range(min(_K, nch)):
        rd(i).start()

    waited = set()
    for i in range(nch):
        s = i % _K
        rd(i).wait()

        base = i * _CR * _LANES
        contains = jnp.logical_and(idx + 1 >= base,
                                   idx < base + _CR * _LANES)

        @pl.when(contains)
        def _fix(s=s, base=base):
            rr = jax.lax.broadcasted_iota(jnp.int32, (_CR, _LANES), 0)
            cc = jax.lax.broadcasted_iota(jnp.int32, (_CR, _LANES), 1)
            flat = base + rr * _LANES + cc
            mask = jnp.logical_or(flat == idx, flat == idx + 1)
            bufs[s] = jnp.where(mask, jnp.float32(0), bufs[s])

        wr(i).start()
        j = i - _WLAG
        if j >= 0 and j + _K < nch:
            wr(j).wait()
            waited.add(j)
            rd(j + _K).start()

    for i in range(nch):
        if i not in waited:
            wr(i).wait()


def kernel(a, idx):
    n = a.shape[0]
    rows = n // _LANES
    idx32 = idx.astype(jnp.int32)
    a2 = a.reshape(rows, _LANES)
    out = pl.pallas_call(
        _ring_copy_kernel,
        out_shape=jax.ShapeDtypeStruct((rows, _LANES), a.dtype),
        in_specs=[
            pl.BlockSpec(memory_space=pltpu.SMEM),
            pl.BlockSpec(memory_space=pltpu.MemorySpace.HBM),
        ],
        out_specs=pl.BlockSpec(memory_space=pltpu.MemorySpace.HBM),
        scratch_shapes=[pltpu.VMEM((_K, _CR, _LANES), jnp.float32)]
        + [pltpu.SemaphoreType.DMA] * (2 * _K),
    )(idx32, a2)
    return out.reshape(n)


# speedup vs baseline: 1.0750x; 1.0053x over previous
import jax
import jax.numpy as jnp
from jax.experimental import pallas as pl
from jax.experimental.pallas import tpu as pltpu

_LANES = 128
_CR = 8192
_K = 12
_WLAG = 4


def _ring_copy_kernel(idx_ref, a_ref, o_ref, bufs, *sems):
    rows = a_ref.shape[0]
    nch = rows // _CR
    rsem = sems[:_K]
    wsem = sems[_K:]
    idx = idx_ref[0]

    def rd(i):
        s = i % _K
        return pltpu.make_async_copy(
            a_ref.at[pl.ds(i * _CR, _CR), :], bufs.at[s], rsem[s])

    def wr(i):
        s = i % _K
        return pltpu.make_async_copy(
            bufs.at[s], o_ref.at[pl.ds(i * _CR, _CR), :], wsem[s])

    for i in range(min(_K, nch)):
        rd(i).start()

    waited = set()
    for i in range(nch):
        s = i % _K
        rd(i).wait()

        base = i * _CR * _LANES
        contains = jnp.logical_and(idx + 1 >= base,
                                   idx < base + _CR * _LANES)

        @pl.when(contains)
        def _fix(s=s, base=base):
            rr = jax.lax.broadcasted_iota(jnp.int32, (_CR, _LANES), 0)
            cc = jax.lax.broadcasted_iota(jnp.int32, (_CR, _LANES), 1)
            flat = base + rr * _LANES + cc
            mask = jnp.logical_or(flat == idx, flat == idx + 1)
            bufs[s] = jnp.where(mask, jnp.float32(0), bufs[s])

        wr(i).start()
        j = i - _WLAG
        if j >= 0 and j + _K < nch:
            wr(j).wait()
            waited.add(j)
            rd(j + _K).start()

    for i in range(nch):
        if i not in waited:
            wr(i).wait()


def kernel(a, idx):
    n = a.shape[0]
    rows = n // _LANES
    idx32 = idx.astype(jnp.int32)
    a2 = a.reshape(rows, _LANES)
    out = pl.pallas_call(
        _ring_copy_kernel,
        out_shape=jax.ShapeDtypeStruct((rows, _LANES), a.dtype),
        in_specs=[
            pl.BlockSpec(memory_space=pltpu.SMEM),
            pl.BlockSpec(memory_space=pltpu.MemorySpace.HBM),
        ],
        out_specs=pl.BlockSpec(memory_space=pltpu.MemorySpace.HBM),
        scratch_shapes=[pltpu.VMEM((_K, _CR, _LANES), jnp.float32)]
        + [pltpu.SemaphoreType.DMA] * (2 * _K),
    )(idx32, a2)
    return out.reshape(n)
